# Initial kernel scaffold; baseline (speedup 1.0000x reference)
import jax
import jax.numpy as jnp
from jax.experimental import pallas as pl


def kernel(inputs, W0, W1, W2, W3, W4, W5):
    raise NotImplementedError("write your pallas kernel here")


# speedup vs baseline: 1.5423x; 1.5423x over previous
import functools

import jax
import jax.numpy as jnp
from jax import lax
from jax.experimental import pallas as pl
from jax.experimental.pallas import tpu as pltpu
from jax.experimental.pallas import tpu_sc as plsc

VOCAB = 100000
DIM = 768
NTAB = 6
B, S = 4, 2048
NIDS = B * S

NC, NS = 2, 16
NW = NC * NS
IDS_PER_W = NIDS // NW
CH = 128
NCHUNK = IDS_PER_W // CH


def _make_gather():
  mesh = plsc.VectorSubcoreMesh(core_axis_name="c", subcore_axis_name="s")

  @functools.partial(
      pl.kernel,
      out_type=tuple(
          jax.ShapeDtypeStruct((NIDS, DIM), jnp.float32) for _ in range(NTAB)
      ),
      mesh=mesh,
      scratch_types=[
          pltpu.VMEM((NCHUNK, CH), jnp.int32),
          pltpu.VMEM((CH, DIM), jnp.float32),
          pltpu.SemaphoreType.DMA,
      ],
  )
  def gather6(idx_hbm, w0, w1, w2, w3, w4, w5, o0, o1, o2, o3, o4, o5,
              idx_v, rows_v, sem):
    wid = lax.axis_index("s") * NC + lax.axis_index("c")
    base = wid * IDS_PER_W
    pltpu.sync_copy(idx_hbm.at[wid], idx_v)
    for w, o in ((w0, o0), (w1, o1), (w2, o2), (w3, o3), (w4, o4), (w5, o5)):
      for c in range(NCHUNK):
        pltpu.async_copy(w.at[idx_v.at[c]], rows_v, sem).wait()
        pltpu.sync_copy(rows_v, o.at[pl.ds(base + c * CH, CH)])

  return gather6


_gather6 = _make_gather()


def kernel(inputs, W0, W1, W2, W3, W4, W5):
  idx = inputs.reshape(NW, NCHUNK, CH)
  outs = _gather6(idx, W0, W1, W2, W3, W4, W5)
  ve = tuple(o.reshape(B, S, DIM) for o in outs)
  return ve + tuple(reversed(ve))


# speedup vs baseline: 1.5686x; 1.0171x over previous
import functools

import jax
import jax.numpy as jnp
from jax import lax
from jax.experimental import pallas as pl
from jax.experimental.pallas import tpu as pltpu
from jax.experimental.pallas import tpu_sc as plsc

VOCAB = 100000
DIM = 768
NTAB = 6
B, S = 4, 2048
NIDS = B * S

NC, NS = 2, 16
NW = NC * NS
IDS_PER_W = NIDS // NW
CH = 64
NCHUNK = IDS_PER_W // CH


def _make_gather():
  mesh = plsc.VectorSubcoreMesh(core_axis_name="c", subcore_axis_name="s")

  @functools.partial(
      pl.kernel,
      out_type=tuple(
          jax.ShapeDtypeStruct((NIDS, DIM), jnp.float32) for _ in range(NTAB)
      ),
      mesh=mesh,
      scratch_types=[
          pltpu.VMEM((NCHUNK, CH), jnp.int32),
          pltpu.VMEM((CH, DIM), jnp.float32),
          pltpu.VMEM((CH, DIM), jnp.float32),
          pltpu.SemaphoreType.DMA,
          pltpu.SemaphoreType.DMA,
          pltpu.SemaphoreType.DMA,
          pltpu.SemaphoreType.DMA,
      ],
  )
  def gather6(idx_hbm, w0, w1, w2, w3, w4, w5, o0, o1, o2, o3, o4, o5,
              idx_v, rows0, rows1, gs0, gs1, ws0, ws1):
    wid = lax.axis_index("s") * NC + lax.axis_index("c")
    base = wid * IDS_PER_W
    pltpu.sync_copy(idx_hbm.at[wid], idx_v)
    bufs = (rows0, rows1)
    gsems = (gs0, gs1)
    wsems = (ws0, ws1)
    work = [
        (w, o, c)
        for w, o in ((w0, o0), (w1, o1), (w2, o2), (w3, o3), (w4, o4), (w5, o5))
        for c in range(NCHUNK)
    ]
    n = len(work)
    gathers = [None] * n
    writes = [None] * n
    w0_, _, c0_ = work[0]
    gathers[0] = pltpu.async_copy(w0_.at[idx_v.at[c0_]], bufs[0], gsems[0])
    for i in range(n):
      b = i % 2
      nb = (i + 1) % 2
      if i + 1 < n:
        if i >= 1:
          writes[i - 1].wait()
        w, _, c = work[i + 1]
        gathers[i + 1] = pltpu.async_copy(w.at[idx_v.at[c]], bufs[nb], gsems[nb])
      gathers[i].wait()
      _, o, c = work[i]
      writes[i] = pltpu.async_copy(bufs[b], o.at[pl.ds(base + c * CH, CH)], wsems[b])
    writes[n - 2].wait()
    writes[n - 1].wait()

  return gather6


_gather6 = _make_gather()


def kernel(inputs, W0, W1, W2, W3, W4, W5):
  idx = inputs.reshape(NW, NCHUNK, CH)
  outs = _gather6(idx, W0, W1, W2, W3, W4, W5)
  ve = tuple(o.reshape(B, S, DIM) for o in outs)
  return ve + tuple(reversed(ve))
